# baseline (speedup 1.0000x reference)
import jax
import jax.numpy as jnp
from jax import lax
from jax.experimental import pallas as pl
from jax.experimental.pallas import tpu as pltpu
from jax.experimental.pallas import tpu_sc as plsc

N_OBS = 800_000
N_CAM = 1000
N_PTS = 200_000
NW = 32
CHUNK = 1280
N_CHUNKS = N_OBS // CHUNK
ITERS = -(-N_CHUNKS // NW)
STEPS = CHUNK // 16
IDX_SUB = 128
N_SUB = CHUNK // IDX_SUB
ROWS_A = 12504
ROWS_LAST = N_PTS - 15 * ROWS_A


def _body(p2d_hbm, cidx_hbm, pidx_hbm, cam_hbm,
          px_hbm, py_hbm, pz_hbm, out_hbm, *scr):
    px_sh, py_sh, pz_sh, cam_tab, s2_tab = scr[0:5]
    slots = [scr[5 + 9 * b:5 + 9 * (b + 1)] for b in range(2)]
    sems = scr[23:29]

    c = lax.axis_index("c")
    s = lax.axis_index("s")
    wid = s * 2 + c

    pltpu.sync_copy(cam_hbm, cam_tab)

    @pl.when(s < 15)
    def _():
        pltpu.sync_copy(px_hbm.at[pl.ds(s * ROWS_A, ROWS_A)],
                        px_sh.at[pl.ds(s * ROWS_A, ROWS_A)])
        pltpu.sync_copy(py_hbm.at[pl.ds(s * ROWS_A, ROWS_A)],
                        py_sh.at[pl.ds(s * ROWS_A, ROWS_A)])
        pltpu.sync_copy(pz_hbm.at[pl.ds(s * ROWS_A, ROWS_A)],
                        pz_sh.at[pl.ds(s * ROWS_A, ROWS_A)])

    @pl.when(s == 15)
    def _():
        pltpu.sync_copy(px_hbm.at[pl.ds(15 * ROWS_A, ROWS_LAST)],
                        px_sh.at[pl.ds(15 * ROWS_A, ROWS_LAST)])
        pltpu.sync_copy(py_hbm.at[pl.ds(15 * ROWS_A, ROWS_LAST)],
                        py_sh.at[pl.ds(15 * ROWS_A, ROWS_LAST)])
        pltpu.sync_copy(pz_hbm.at[pl.ds(15 * ROWS_A, ROWS_LAST)],
                        pz_sh.at[pl.ds(15 * ROWS_A, ROWS_LAST)])

    plsc.subcore_barrier()

    @plsc.parallel_loop(0, (N_CAM + 15) // 16, 1)
    def _init_s2(i):
        sl = pl.ds(i * 16, 16)
        qw = cam_tab[sl]
        qx = cam_tab[pl.ds(N_CAM + i * 16, 16)]
        qy = cam_tab[pl.ds(2 * N_CAM + i * 16, 16)]
        qz = cam_tab[pl.ds(3 * N_CAM + i * 16, 16)]
        s2_tab[sl] = 2.0 / (qw * qw + qx * qx + qy * qy + qz * qz)

    cam_off = [jnp.full((16,), j * N_CAM, jnp.int32) for j in range(10)]

    def issue_linear(cid, b):
        base = cid * CHUNK
        S = slots[b]
        sem = sems[3 * b]
        pltpu.async_copy(cidx_hbm.at[pl.ds(base, CHUNK)], S[0], sem)
        pltpu.async_copy(pidx_hbm.at[pl.ds(base, CHUNK)], S[1], sem)
        pltpu.async_copy(p2d_hbm.at[pl.ds(base, CHUNK)], S[2], sem)
        pltpu.async_copy(p2d_hbm.at[pl.ds(N_OBS + base, CHUNK)], S[3], sem)

    def wait_linear(b):
        S = slots[b]
        sem = sems[3 * b]
        pltpu.make_async_copy(cidx_hbm.at[pl.ds(0, CHUNK)], S[0], sem).wait()
        pltpu.make_async_copy(pidx_hbm.at[pl.ds(0, CHUNK)], S[1], sem).wait()
        pltpu.make_async_copy(p2d_hbm.at[pl.ds(0, CHUNK)], S[2], sem).wait()
        pltpu.make_async_copy(p2d_hbm.at[pl.ds(0, CHUNK)], S[3], sem).wait()

    def issue_gathers(b):
        S = slots[b]
        sem = sems[3 * b + 1]
        for j in range(N_SUB):
            ids = S[1].at[pl.ds(j * IDX_SUB, IDX_SUB)]
            dst = pl.ds(j * IDX_SUB, IDX_SUB)
            pltpu.async_copy(px_sh.at[ids], S[4].at[dst], sem)
            pltpu.async_copy(py_sh.at[ids], S[5].at[dst], sem)
            pltpu.async_copy(pz_sh.at[ids], S[6].at[dst], sem)

    def wait_gathers(b):
        S = slots[b]
        sem = sems[3 * b + 1]
        for j in range(N_SUB):
            ids = S[1].at[pl.ds(j * IDX_SUB, IDX_SUB)]
            dst = pl.ds(j * IDX_SUB, IDX_SUB)
            pltpu.make_async_copy(px_sh.at[ids], S[4].at[dst], sem).wait()
            pltpu.make_async_copy(py_sh.at[ids], S[5].at[dst], sem).wait()
            pltpu.make_async_copy(pz_sh.at[ids], S[6].at[dst], sem).wait()

    def issue_out(cid, b):
        base = cid * CHUNK
        S = slots[b]
        sem = sems[3 * b + 2]
        pltpu.async_copy(S[7], out_hbm.at[pl.ds(base, CHUNK)], sem)
        pltpu.async_copy(S[8], out_hbm.at[pl.ds(N_OBS + base, CHUNK)], sem)

    def wait_out(b):
        S = slots[b]
        sem = sems[3 * b + 2]
        pltpu.make_async_copy(S[7], out_hbm.at[pl.ds(0, CHUNK)], sem).wait()
        pltpu.make_async_copy(S[8], out_hbm.at[pl.ds(0, CHUNK)], sem).wait()

    def compute(b):
        _, _, u2_v, v2_v, pxv, pyv, pzv, ou_v, ov_v = slots[b][:9]
        cidx_v = slots[b][0]

        @plsc.parallel_loop(0, STEPS, 1, unroll=6)
        def step(i):
            sl = pl.ds(i * 16, 16)
            ci = cidx_v[sl]
            qw = plsc.load_gather(cam_tab, [ci + cam_off[0]])
            qx = plsc.load_gather(cam_tab, [ci + cam_off[1]])
            qy = plsc.load_gather(cam_tab, [ci + cam_off[2]])
            qz = plsc.load_gather(cam_tab, [ci + cam_off[3]])
            trx = plsc.load_gather(cam_tab, [ci + cam_off[4]])
            try_ = plsc.load_gather(cam_tab, [ci + cam_off[5]])
            trz = plsc.load_gather(cam_tab, [ci + cam_off[6]])
            f = plsc.load_gather(cam_tab, [ci + cam_off[7]])
            k1 = plsc.load_gather(cam_tab, [ci + cam_off[8]])
            k2 = plsc.load_gather(cam_tab, [ci + cam_off[9]])
            px = pxv[sl]
            py = pyv[sl]
            pz = pzv[sl]

            s2 = plsc.load_gather(s2_tab, [ci])
            tx = qy * pz - qz * py + qw * px
            ty = qz * px - qx * pz + qw * py
            tz = qx * py - qy * px + qw * pz
            rx = px + s2 * (qy * tz - qz * ty) + trx
            ry = py + s2 * (qz * tx - qx * tz) + try_
            rz = pz + s2 * (qx * ty - qy * tx) + trz
            inv_rz = -1.0 / rz
            u = rx * inv_rz
            v = ry * inv_rz
            n = u * u + v * v
            r = 1.0 + k1 * n + k2 * n * n
            fr = f * r
            ou_v[sl] = u * fr - u2_v[sl]
            ov_v[sl] = v * fr - v2_v[sl]

    issue_linear(wid, 0)
    wait_linear(0)
    issue_gathers(0)
    issue_linear(wid + NW, 1)

    def outer(j, carry):
        for b in range(2):
            k = j * 2 + b
            cid = wid + k * NW

            @pl.when(cid + NW < N_CHUNKS)
            def _():
                wait_linear(1 - b)
                issue_gathers(1 - b)

            @pl.when(cid < N_CHUNKS)
            def _():
                wait_gathers(b)

                @pl.when(cid >= 2 * NW)
                def _():
                    wait_out(b)

                compute(b)
                issue_out(cid, b)

            @pl.when(cid + 2 * NW < N_CHUNKS)
            def _():
                issue_linear(cid + 2 * NW, b)

        return carry

    lax.fori_loop(0, ITERS // 2, outer, 0)

    wait_out(0)
    wait_out(1)


_mesh = plsc.VectorSubcoreMesh(core_axis_name="c", subcore_axis_name="s")

_slot_bufs = [
    pltpu.VMEM((CHUNK,), jnp.int32),
    pltpu.VMEM((CHUNK,), jnp.int32),
    pltpu.VMEM((CHUNK,), jnp.float32),
    pltpu.VMEM((CHUNK,), jnp.float32),
    pltpu.VMEM((CHUNK,), jnp.float32),
    pltpu.VMEM((CHUNK,), jnp.float32),
    pltpu.VMEM((CHUNK,), jnp.float32),
    pltpu.VMEM((CHUNK,), jnp.float32),
    pltpu.VMEM((CHUNK,), jnp.float32),
]

_proj = pl.kernel(
    _body,
    out_type=jax.ShapeDtypeStruct((2 * N_OBS,), jnp.float32),
    mesh=_mesh,
    scratch_types=(
        [
            pltpu.VMEM_SHARED((N_PTS,), jnp.float32),
            pltpu.VMEM_SHARED((N_PTS,), jnp.float32),
            pltpu.VMEM_SHARED((N_PTS,), jnp.float32),
            pltpu.VMEM((10 * N_CAM,), jnp.float32),
            pltpu.VMEM((16 * ((N_CAM + 15) // 16),), jnp.float32),
        ]
        + _slot_bufs + _slot_bufs
        + [pltpu.SemaphoreType.DMA] * 6
    ),
    compiler_params=pltpu.CompilerParams(
        needs_layout_passes=False, use_tc_tiling_on_sc=False),
)


def kernel(points_2d, camera_indices, point_indices, camera_params, points_3d):
    p2d_flat = points_2d.T.reshape(-1)
    ci = camera_indices.astype(jnp.int32)
    pi = point_indices.astype(jnp.int32)
    cam_t = camera_params.T.reshape(-1)
    px = points_3d[:, 0]
    py = points_3d[:, 1]
    pz = points_3d[:, 2]
    out_flat = _proj(p2d_flat, ci, pi, cam_t, px, py, pz)
    return out_flat.reshape(2, N_OBS).T


# speedup vs baseline: 1.1946x; 1.1946x over previous
import jax
import jax.numpy as jnp
from jax import lax
from jax.experimental import pallas as pl
from jax.experimental.pallas import tpu as pltpu
from jax.experimental.pallas import tpu_sc as plsc

N_OBS = 800_000
N_CAM = 1000
N_PTS = 200_000
NW = 32
CHUNK = 1280
N_CHUNKS = N_OBS // CHUNK
ITERS = -(-N_CHUNKS // NW)
STEPS = CHUNK // 16
IDX_SUB = 128
N_SUB = CHUNK // IDX_SUB
ROWS_A = 12504
ROWS_LAST = N_PTS - 15 * ROWS_A


def _body(p2d_hbm, cidx_hbm, pidx_hbm, cam_hbm,
          px_hbm, py_hbm, pz_hbm, out_hbm, *scr):
    px_sh, py_sh, pz_sh, cam_tab, s2_tab = scr[0:5]
    slots = [scr[5 + 9 * b:5 + 9 * (b + 1)] for b in range(2)]
    sems = scr[23:29]

    c = lax.axis_index("c")
    s = lax.axis_index("s")
    wid = s * 2 + c

    pltpu.sync_copy(cam_hbm, cam_tab)

    @pl.when(s < 15)
    def _():
        pltpu.sync_copy(px_hbm.at[pl.ds(s * ROWS_A, ROWS_A)],
                        px_sh.at[pl.ds(s * ROWS_A, ROWS_A)])
        pltpu.sync_copy(py_hbm.at[pl.ds(s * ROWS_A, ROWS_A)],
                        py_sh.at[pl.ds(s * ROWS_A, ROWS_A)])
        pltpu.sync_copy(pz_hbm.at[pl.ds(s * ROWS_A, ROWS_A)],
                        pz_sh.at[pl.ds(s * ROWS_A, ROWS_A)])

    @pl.when(s == 15)
    def _():
        pltpu.sync_copy(px_hbm.at[pl.ds(15 * ROWS_A, ROWS_LAST)],
                        px_sh.at[pl.ds(15 * ROWS_A, ROWS_LAST)])
        pltpu.sync_copy(py_hbm.at[pl.ds(15 * ROWS_A, ROWS_LAST)],
                        py_sh.at[pl.ds(15 * ROWS_A, ROWS_LAST)])
        pltpu.sync_copy(pz_hbm.at[pl.ds(15 * ROWS_A, ROWS_LAST)],
                        pz_sh.at[pl.ds(15 * ROWS_A, ROWS_LAST)])

    plsc.subcore_barrier()

    @plsc.parallel_loop(0, (N_CAM + 15) // 16, 1)
    def _init_s2(i):
        sl = pl.ds(i * 16, 16)
        qw = cam_tab[sl]
        qx = cam_tab[pl.ds(N_CAM + i * 16, 16)]
        qy = cam_tab[pl.ds(2 * N_CAM + i * 16, 16)]
        qz = cam_tab[pl.ds(3 * N_CAM + i * 16, 16)]
        s2_tab[sl] = 2.0 / (qw * qw + qx * qx + qy * qy + qz * qz)

    cam_off = [jnp.full((16,), j * N_CAM, jnp.int32) for j in range(10)]

    def issue_linear(cid, b):
        base = cid * CHUNK
        S = slots[b]
        sem = sems[3 * b]
        pltpu.async_copy(cidx_hbm.at[pl.ds(base, CHUNK)], S[0], sem)
        pltpu.async_copy(pidx_hbm.at[pl.ds(base, CHUNK)], S[1], sem)
        pltpu.async_copy(p2d_hbm.at[pl.ds(base, CHUNK)], S[2], sem)
        pltpu.async_copy(p2d_hbm.at[pl.ds(N_OBS + base, CHUNK)], S[3], sem)

    def wait_linear(b):
        S = slots[b]
        sem = sems[3 * b]
        pltpu.make_async_copy(cidx_hbm.at[pl.ds(0, CHUNK)], S[0], sem).wait()
        pltpu.make_async_copy(pidx_hbm.at[pl.ds(0, CHUNK)], S[1], sem).wait()
        pltpu.make_async_copy(p2d_hbm.at[pl.ds(0, CHUNK)], S[2], sem).wait()
        pltpu.make_async_copy(p2d_hbm.at[pl.ds(0, CHUNK)], S[3], sem).wait()

    def issue_gathers(b):
        S = slots[b]
        sem = sems[3 * b + 1]
        for j in range(N_SUB):
            ids = S[1].at[pl.ds(j * IDX_SUB, IDX_SUB)]
            dst = pl.ds(j * IDX_SUB, IDX_SUB)
            pltpu.async_copy(px_sh.at[ids], S[4].at[dst], sem)
            pltpu.async_copy(py_sh.at[ids], S[5].at[dst], sem)
            pltpu.async_copy(pz_sh.at[ids], S[6].at[dst], sem)

    def wait_gathers(b):
        S = slots[b]
        sem = sems[3 * b + 1]
        for j in range(N_SUB):
            ids = S[1].at[pl.ds(j * IDX_SUB, IDX_SUB)]
            dst = pl.ds(j * IDX_SUB, IDX_SUB)
            pltpu.make_async_copy(px_sh.at[ids], S[4].at[dst], sem).wait()
            pltpu.make_async_copy(py_sh.at[ids], S[5].at[dst], sem).wait()
            pltpu.make_async_copy(pz_sh.at[ids], S[6].at[dst], sem).wait()

    def issue_out(cid, b):
        base = cid * CHUNK
        S = slots[b]
        sem = sems[3 * b + 2]
        pltpu.async_copy(S[7], out_hbm.at[pl.ds(base, CHUNK)], sem)
        pltpu.async_copy(S[8], out_hbm.at[pl.ds(N_OBS + base, CHUNK)], sem)

    def wait_out(b):
        S = slots[b]
        sem = sems[3 * b + 2]
        pltpu.make_async_copy(S[7], out_hbm.at[pl.ds(0, CHUNK)], sem).wait()
        pltpu.make_async_copy(S[8], out_hbm.at[pl.ds(0, CHUNK)], sem).wait()

    def compute(b):
        _, _, u2_v, v2_v, pxv, pyv, pzv, ou_v, ov_v = slots[b][:9]
        cidx_v = slots[b][0]

        @plsc.parallel_loop(0, STEPS, 1, unroll=2)
        def step(i):
            sl = pl.ds(i * 16, 16)
            ci = cidx_v[sl]
            qw = plsc.load_gather(cam_tab, [ci + cam_off[0]])
            qx = plsc.load_gather(cam_tab, [ci + cam_off[1]])
            qy = plsc.load_gather(cam_tab, [ci + cam_off[2]])
            qz = plsc.load_gather(cam_tab, [ci + cam_off[3]])
            trx = plsc.load_gather(cam_tab, [ci + cam_off[4]])
            try_ = plsc.load_gather(cam_tab, [ci + cam_off[5]])
            trz = plsc.load_gather(cam_tab, [ci + cam_off[6]])
            f = plsc.load_gather(cam_tab, [ci + cam_off[7]])
            k1 = plsc.load_gather(cam_tab, [ci + cam_off[8]])
            k2 = plsc.load_gather(cam_tab, [ci + cam_off[9]])
            px = pxv[sl]
            py = pyv[sl]
            pz = pzv[sl]

            s2 = plsc.load_gather(s2_tab, [ci])
            tx = qy * pz - qz * py + qw * px
            ty = qz * px - qx * pz + qw * py
            tz = qx * py - qy * px + qw * pz
            rx = px + s2 * (qy * tz - qz * ty) + trx
            ry = py + s2 * (qz * tx - qx * tz) + try_
            rz = pz + s2 * (qx * ty - qy * tx) + trz
            inv_rz = -1.0 / rz
            u = rx * inv_rz
            v = ry * inv_rz
            n = u * u + v * v
            r = 1.0 + k1 * n + k2 * n * n
            fr = f * r
            ou_v[sl] = u * fr - u2_v[sl]
            ov_v[sl] = v * fr - v2_v[sl]

    issue_linear(wid, 0)
    wait_linear(0)
    issue_gathers(0)
    issue_linear(wid + NW, 1)

    def outer(j, carry):
        for b in range(2):
            k = j * 2 + b
            cid = wid + k * NW

            @pl.when(cid + NW < N_CHUNKS)
            def _():
                wait_linear(1 - b)
                issue_gathers(1 - b)

            @pl.when(cid < N_CHUNKS)
            def _():
                wait_gathers(b)

                @pl.when(cid >= 2 * NW)
                def _():
                    wait_out(b)

                compute(b)
                issue_out(cid, b)

            @pl.when(cid + 2 * NW < N_CHUNKS)
            def _():
                issue_linear(cid + 2 * NW, b)

        return carry

    lax.fori_loop(0, ITERS // 2, outer, 0)

    wait_out(0)
    wait_out(1)


_mesh = plsc.VectorSubcoreMesh(core_axis_name="c", subcore_axis_name="s")

_slot_bufs = [
    pltpu.VMEM((CHUNK,), jnp.int32),
    pltpu.VMEM((CHUNK,), jnp.int32),
    pltpu.VMEM((CHUNK,), jnp.float32),
    pltpu.VMEM((CHUNK,), jnp.float32),
    pltpu.VMEM((CHUNK,), jnp.float32),
    pltpu.VMEM((CHUNK,), jnp.float32),
    pltpu.VMEM((CHUNK,), jnp.float32),
    pltpu.VMEM((CHUNK,), jnp.float32),
    pltpu.VMEM((CHUNK,), jnp.float32),
]

_proj = pl.kernel(
    _body,
    out_type=jax.ShapeDtypeStruct((2 * N_OBS,), jnp.float32),
    mesh=_mesh,
    scratch_types=(
        [
            pltpu.VMEM_SHARED((N_PTS,), jnp.float32),
            pltpu.VMEM_SHARED((N_PTS,), jnp.float32),
            pltpu.VMEM_SHARED((N_PTS,), jnp.float32),
            pltpu.VMEM((10 * N_CAM,), jnp.float32),
            pltpu.VMEM((16 * ((N_CAM + 15) // 16),), jnp.float32),
        ]
        + _slot_bufs + _slot_bufs
        + [pltpu.SemaphoreType.DMA] * 6
    ),
    compiler_params=pltpu.CompilerParams(
        needs_layout_passes=False, use_tc_tiling_on_sc=False),
)


def kernel(points_2d, camera_indices, point_indices, camera_params, points_3d):
    p2d_flat = points_2d.T.reshape(-1)
    ci = camera_indices.astype(jnp.int32)
    pi = point_indices.astype(jnp.int32)
    cam_t = camera_params.T.reshape(-1)
    px = points_3d[:, 0]
    py = points_3d[:, 1]
    pz = points_3d[:, 2]
    out_flat = _proj(p2d_flat, ci, pi, cam_t, px, py, pz)
    return out_flat.reshape(2, N_OBS).T


# speedup vs baseline: 1.1972x; 1.0021x over previous
import jax
import jax.numpy as jnp
from jax import lax
from jax.experimental import pallas as pl
from jax.experimental.pallas import tpu as pltpu
from jax.experimental.pallas import tpu_sc as plsc

N_OBS = 800_000
N_CAM = 1000
N_PTS = 200_000
NW = 32
CHUNK = 1280
N_CHUNKS = N_OBS // CHUNK
ITERS = -(-N_CHUNKS // NW)
STEPS = CHUNK // 16
IDX_SUB = 128
N_SUB = CHUNK // IDX_SUB
ROWS_A = 12504
ROWS_LAST = N_PTS - 15 * ROWS_A


def _body(p2d_hbm, cidx_hbm, pidx_hbm, cam_hbm,
          px_hbm, py_hbm, pz_hbm, out_hbm, *scr):
    px_sh, py_sh, pz_sh, cam_tab, s2_tab = scr[0:5]
    slots = [scr[5 + 9 * b:5 + 9 * (b + 1)] for b in range(2)]
    sems = scr[23:29]

    c = lax.axis_index("c")
    s = lax.axis_index("s")
    wid = s * 2 + c

    pltpu.sync_copy(cam_hbm, cam_tab)

    @pl.when(s < 15)
    def _():
        pltpu.sync_copy(px_hbm.at[pl.ds(s * ROWS_A, ROWS_A)],
                        px_sh.at[pl.ds(s * ROWS_A, ROWS_A)])
        pltpu.sync_copy(py_hbm.at[pl.ds(s * ROWS_A, ROWS_A)],
                        py_sh.at[pl.ds(s * ROWS_A, ROWS_A)])
        pltpu.sync_copy(pz_hbm.at[pl.ds(s * ROWS_A, ROWS_A)],
                        pz_sh.at[pl.ds(s * ROWS_A, ROWS_A)])

    @pl.when(s == 15)
    def _():
        pltpu.sync_copy(px_hbm.at[pl.ds(15 * ROWS_A, ROWS_LAST)],
                        px_sh.at[pl.ds(15 * ROWS_A, ROWS_LAST)])
        pltpu.sync_copy(py_hbm.at[pl.ds(15 * ROWS_A, ROWS_LAST)],
                        py_sh.at[pl.ds(15 * ROWS_A, ROWS_LAST)])
        pltpu.sync_copy(pz_hbm.at[pl.ds(15 * ROWS_A, ROWS_LAST)],
                        pz_sh.at[pl.ds(15 * ROWS_A, ROWS_LAST)])

    plsc.subcore_barrier()

    @plsc.parallel_loop(0, (N_CAM + 15) // 16, 1)
    def _init_s2(i):
        sl = pl.ds(i * 16, 16)
        qw = cam_tab[sl]
        qx = cam_tab[pl.ds(N_CAM + i * 16, 16)]
        qy = cam_tab[pl.ds(2 * N_CAM + i * 16, 16)]
        qz = cam_tab[pl.ds(3 * N_CAM + i * 16, 16)]
        s2_tab[sl] = 2.0 / (qw * qw + qx * qx + qy * qy + qz * qz)

    cam_off = [jnp.full((16,), j * N_CAM, jnp.int32) for j in range(10)]

    def issue_linear(cid, b):
        base = cid * CHUNK
        S = slots[b]
        sem = sems[3 * b]
        pltpu.async_copy(cidx_hbm.at[pl.ds(base, CHUNK)], S[0], sem)
        pltpu.async_copy(pidx_hbm.at[pl.ds(base, CHUNK)], S[1], sem)
        pltpu.async_copy(p2d_hbm.at[pl.ds(base, CHUNK)], S[2], sem)
        pltpu.async_copy(p2d_hbm.at[pl.ds(N_OBS + base, CHUNK)], S[3], sem)

    def wait_linear(b):
        S = slots[b]
        sem = sems[3 * b]
        pltpu.make_async_copy(cidx_hbm.at[pl.ds(0, CHUNK)], S[0], sem).wait()
        pltpu.make_async_copy(pidx_hbm.at[pl.ds(0, CHUNK)], S[1], sem).wait()
        pltpu.make_async_copy(p2d_hbm.at[pl.ds(0, CHUNK)], S[2], sem).wait()
        pltpu.make_async_copy(p2d_hbm.at[pl.ds(0, CHUNK)], S[3], sem).wait()

    def issue_gathers(b):
        S = slots[b]
        sem = sems[3 * b + 1]
        for j in range(N_SUB):
            ids = S[1].at[pl.ds(j * IDX_SUB, IDX_SUB)]
            dst = pl.ds(j * IDX_SUB, IDX_SUB)
            pltpu.async_copy(px_sh.at[ids], S[4].at[dst], sem)
            pltpu.async_copy(py_sh.at[ids], S[5].at[dst], sem)
            pltpu.async_copy(pz_sh.at[ids], S[6].at[dst], sem)

    def wait_gathers(b):
        S = slots[b]
        sem = sems[3 * b + 1]
        for j in range(N_SUB):
            ids = S[1].at[pl.ds(j * IDX_SUB, IDX_SUB)]
            dst = pl.ds(j * IDX_SUB, IDX_SUB)
            pltpu.make_async_copy(px_sh.at[ids], S[4].at[dst], sem).wait()
            pltpu.make_async_copy(py_sh.at[ids], S[5].at[dst], sem).wait()
            pltpu.make_async_copy(pz_sh.at[ids], S[6].at[dst], sem).wait()

    def issue_out(cid, b):
        base = cid * CHUNK
        S = slots[b]
        sem = sems[3 * b + 2]
        pltpu.async_copy(S[7], out_hbm.at[pl.ds(base, CHUNK)], sem)
        pltpu.async_copy(S[8], out_hbm.at[pl.ds(N_OBS + base, CHUNK)], sem)

    def wait_out(b):
        S = slots[b]
        sem = sems[3 * b + 2]
        pltpu.make_async_copy(S[7], out_hbm.at[pl.ds(0, CHUNK)], sem).wait()
        pltpu.make_async_copy(S[8], out_hbm.at[pl.ds(0, CHUNK)], sem).wait()

    def compute(b):
        _, _, u2_v, v2_v, pxv, pyv, pzv, ou_v, ov_v = slots[b][:9]
        cidx_v = slots[b][0]

        @plsc.parallel_loop(0, STEPS, 1, unroll=1)
        def step(i):
            sl = pl.ds(i * 16, 16)
            ci = cidx_v[sl]
            qw = plsc.load_gather(cam_tab, [ci + cam_off[0]])
            qx = plsc.load_gather(cam_tab, [ci + cam_off[1]])
            qy = plsc.load_gather(cam_tab, [ci + cam_off[2]])
            qz = plsc.load_gather(cam_tab, [ci + cam_off[3]])
            trx = plsc.load_gather(cam_tab, [ci + cam_off[4]])
            try_ = plsc.load_gather(cam_tab, [ci + cam_off[5]])
            trz = plsc.load_gather(cam_tab, [ci + cam_off[6]])
            f = plsc.load_gather(cam_tab, [ci + cam_off[7]])
            k1 = plsc.load_gather(cam_tab, [ci + cam_off[8]])
            k2 = plsc.load_gather(cam_tab, [ci + cam_off[9]])
            px = pxv[sl]
            py = pyv[sl]
            pz = pzv[sl]

            s2 = plsc.load_gather(s2_tab, [ci])
            tx = qy * pz - qz * py + qw * px
            ty = qz * px - qx * pz + qw * py
            tz = qx * py - qy * px + qw * pz
            rx = px + s2 * (qy * tz - qz * ty) + trx
            ry = py + s2 * (qz * tx - qx * tz) + try_
            rz = pz + s2 * (qx * ty - qy * tx) + trz
            inv_rz = -1.0 / rz
            u = rx * inv_rz
            v = ry * inv_rz
            n = u * u + v * v
            r = 1.0 + k1 * n + k2 * n * n
            fr = f * r
            ou_v[sl] = u * fr - u2_v[sl]
            ov_v[sl] = v * fr - v2_v[sl]

    issue_linear(wid, 0)
    wait_linear(0)
    issue_gathers(0)
    issue_linear(wid + NW, 1)

    def outer(j, carry):
        for b in range(2):
            k = j * 2 + b
            cid = wid + k * NW

            @pl.when(cid + NW < N_CHUNKS)
            def _():
                wait_linear(1 - b)
                issue_gathers(1 - b)

            @pl.when(cid < N_CHUNKS)
            def _():
                wait_gathers(b)

                @pl.when(cid >= 2 * NW)
                def _():
                    wait_out(b)

                compute(b)
                issue_out(cid, b)

            @pl.when(cid + 2 * NW < N_CHUNKS)
            def _():
                issue_linear(cid + 2 * NW, b)

        return carry

    lax.fori_loop(0, ITERS // 2, outer, 0)

    wait_out(0)
    wait_out(1)


_mesh = plsc.VectorSubcoreMesh(core_axis_name="c", subcore_axis_name="s")

_slot_bufs = [
    pltpu.VMEM((CHUNK,), jnp.int32),
    pltpu.VMEM((CHUNK,), jnp.int32),
    pltpu.VMEM((CHUNK,), jnp.float32),
    pltpu.VMEM((CHUNK,), jnp.float32),
    pltpu.VMEM((CHUNK,), jnp.float32),
    pltpu.VMEM((CHUNK,), jnp.float32),
    pltpu.VMEM((CHUNK,), jnp.float32),
    pltpu.VMEM((CHUNK,), jnp.float32),
    pltpu.VMEM((CHUNK,), jnp.float32),
]

_proj = pl.kernel(
    _body,
    out_type=jax.ShapeDtypeStruct((2 * N_OBS,), jnp.float32),
    mesh=_mesh,
    scratch_types=(
        [
            pltpu.VMEM_SHARED((N_PTS,), jnp.float32),
            pltpu.VMEM_SHARED((N_PTS,), jnp.float32),
            pltpu.VMEM_SHARED((N_PTS,), jnp.float32),
            pltpu.VMEM((10 * N_CAM,), jnp.float32),
            pltpu.VMEM((16 * ((N_CAM + 15) // 16),), jnp.float32),
        ]
        + _slot_bufs + _slot_bufs
        + [pltpu.SemaphoreType.DMA] * 6
    ),
    compiler_params=pltpu.CompilerParams(
        needs_layout_passes=False, use_tc_tiling_on_sc=False),
)


def kernel(points_2d, camera_indices, point_indices, camera_params, points_3d):
    p2d_flat = points_2d.T.reshape(-1)
    ci = camera_indices.astype(jnp.int32)
    pi = point_indices.astype(jnp.int32)
    cam_t = camera_params.T.reshape(-1)
    px = points_3d[:, 0]
    py = points_3d[:, 1]
    pz = points_3d[:, 2]
    out_flat = _proj(p2d_flat, ci, pi, cam_t, px, py, pz)
    return out_flat.reshape(2, N_OBS).T
